# baseline (speedup 1.0000x reference)
import functools

import jax
import jax.numpy as jnp
from jax import lax
from jax.experimental import pallas as pl
from jax.experimental.pallas import tpu as pltpu
from jax.experimental.pallas import tpu_sc as plsc

B, L, V, D = 4096, 200, 1000000, 32
LANES = 16
NUM_WORKERS = 32
BPW = B // NUM_WORKERS
CH = 40
NCH = L // CH
UNROLL = 4


def _num_chunks(len_v, b):
    n = len_v[pl.ds(b, LANES)][0]
    return n, (n + CH - 1) // CH


def _fire_batch(w_hbm, idx_all, rows, sem, len_v, b):
    _, k = _num_chunks(len_v, b)

    def fire(j, carry):
        off = pl.multiple_of(j * CH, CH)
        pltpu.async_copy(
            w_hbm.at[idx_all.at[b, pl.ds(off, CH)]],
            rows.at[pl.ds(off, CH)], sem)
        return carry

    lax.fori_loop(0, k, fire, 0)


def _drain_batch(w_hbm, idx_all, rows, sem, len_v, b):
    _, k = _num_chunks(len_v, b)

    def drain(j, carry):
        off = pl.multiple_of(j * CH, CH)
        pltpu.make_async_copy(
            w_hbm.at[idx_all.at[b, pl.ds(off, CH)]],
            rows.at[pl.ds(off, CH)], sem).wait()
        return carry

    lax.fori_loop(0, k, drain, 0)


def _accum_batch(rows, len_v, out_v, b):
    n = len_v[pl.ds(b, LANES)][0]
    n4 = n >> 2
    r = n & 3

    def body4(i, accs):
        a0, a1 = accs
        base = i * UNROLL
        for j in range(UNROLL):
            a0 = a0 + rows[base + j, pl.ds(0, LANES)]
            a1 = a1 + rows[base + j, pl.ds(LANES, LANES)]
        return a0, a1

    zero = jnp.zeros((LANES,), jnp.float32)
    a0, a1 = lax.fori_loop(0, n4, body4, (zero, zero))
    tail = n4 * UNROLL
    rvec = jnp.full((LANES,), r, dtype=jnp.int32)
    for j in range(UNROLL - 1):
        jv = jnp.full((LANES,), j, dtype=jnp.int32)
        mf = jnp.minimum(jnp.maximum(rvec - jv, 0), 1).astype(jnp.float32)
        a0 = a0 + rows[tail + j, pl.ds(0, LANES)] * mf
        a1 = a1 + rows[tail + j, pl.ds(LANES, LANES)] * mf
    nvec = jnp.full((LANES,), n, dtype=jnp.int32).astype(jnp.float32)
    out_v[b, pl.ds(0, LANES)] = a0 / nvec
    out_v[b, pl.ds(LANES, LANES)] = a1 / nvec


def _mean_embed_body(xs_hbm, len_hbm, w_hbm, out_hbm,
                     idx_all, rows_a, rows_b, len_v, out_v, sem_a, sem_b):
    c = lax.axis_index("c")
    s = lax.axis_index("s")
    wid = s * 2 + c
    base = wid * BPW

    pltpu.sync_copy(len_hbm.at[pl.ds(base, BPW)], len_v.at[pl.ds(0, BPW)])
    pltpu.sync_copy(xs_hbm.at[pl.ds(base, BPW), :], idx_all)

    _fire_batch(w_hbm, idx_all, rows_a, sem_a, len_v, 0)
    _fire_batch(w_hbm, idx_all, rows_b, sem_b, len_v, 1)

    def pair_body(k, carry):
        b0 = 2 * k
        _drain_batch(w_hbm, idx_all, rows_a, sem_a, len_v, b0)
        _accum_batch(rows_a, len_v, out_v, b0)
        _fire_batch(w_hbm, idx_all, rows_a, sem_a, len_v, b0 + 2)
        _drain_batch(w_hbm, idx_all, rows_b, sem_b, len_v, b0 + 1)
        _accum_batch(rows_b, len_v, out_v, b0 + 1)
        _fire_batch(w_hbm, idx_all, rows_b, sem_b, len_v, b0 + 3)
        return carry

    lax.fori_loop(0, BPW // 2 - 1, pair_body, 0)

    _drain_batch(w_hbm, idx_all, rows_a, sem_a, len_v, BPW - 2)
    _accum_batch(rows_a, len_v, out_v, BPW - 2)
    _drain_batch(w_hbm, idx_all, rows_b, sem_b, len_v, BPW - 1)
    _accum_batch(rows_b, len_v, out_v, BPW - 1)

    pltpu.sync_copy(out_v, out_hbm.at[pl.ds(base, BPW)])


@functools.partial(jax.jit, donate_argnums=())
def kernel(xs, xs_len, weight):
    mesh = plsc.VectorSubcoreMesh(core_axis_name="c", subcore_axis_name="s")
    k = functools.partial(
        pl.kernel,
        mesh=mesh,
        compiler_params=pltpu.CompilerParams(use_tc_tiling_on_sc=False),
        out_type=jax.ShapeDtypeStruct((B, D), jnp.float32),
        scratch_types=[
            pltpu.VMEM((BPW, L), jnp.int32),
            pltpu.VMEM((L, D), jnp.float32),
            pltpu.VMEM((L, D), jnp.float32),
            pltpu.VMEM((BPW + LANES,), jnp.int32),
            pltpu.VMEM((BPW, D), jnp.float32),
            pltpu.SemaphoreType.DMA,
            pltpu.SemaphoreType.DMA,
        ],
    )(_mean_embed_body)
    wshield = jnp.minimum(jnp.maximum(weight, jnp.float32(-3.0e38)),
                          jnp.float32(3.0e38))
    return k(xs.astype(jnp.int32), xs_len.astype(jnp.int32), wshield)


# speedup vs baseline: 1.5579x; 1.5579x over previous
import functools

import jax
import jax.numpy as jnp
from jax import lax
from jax.experimental import pallas as pl
from jax.experimental.pallas import tpu as pltpu
from jax.experimental.pallas import tpu_sc as plsc

B, L, V, D = 4096, 200, 1000000, 32
LANES = 16
NUM_WORKERS = 32
BPW = B // NUM_WORKERS
CH = 40
NCH = L // CH
UNROLL = 4


def _num_chunks(len_v, b):
    n = len_v[pl.ds(b, LANES)][0]
    return n, (n + CH - 1) // CH


def _fire_batch(w_hbm, idx_all, rows, sem, len_v, b):
    _, k = _num_chunks(len_v, b)

    def fire(j, carry):
        off = pl.multiple_of(j * CH, CH)
        pltpu.async_copy(
            w_hbm.at[idx_all.at[b, pl.ds(off, CH)]],
            rows.at[pl.ds(off, CH)], sem)
        return carry

    lax.fori_loop(0, k, fire, 0)


def _drain_batch(w_hbm, idx_all, rows, sem, len_v, b):
    _, k = _num_chunks(len_v, b)

    def drain(j, carry):
        off = pl.multiple_of(j * CH, CH)
        pltpu.make_async_copy(
            w_hbm.at[idx_all.at[b, pl.ds(off, CH)]],
            rows.at[pl.ds(off, CH)], sem).wait()
        return carry

    lax.fori_loop(0, k, drain, 0)


def _accum_batch(rows, len_v, out_v, b):
    n = len_v[pl.ds(b, LANES)][0]
    n4 = n >> 2
    r = n & 3

    def body4(i, accs):
        a0, a1 = accs
        base = i * UNROLL
        for j in range(UNROLL):
            a0 = a0 + rows[base + j, pl.ds(0, LANES)]
            a1 = a1 + rows[base + j, pl.ds(LANES, LANES)]
        return a0, a1

    zero = jnp.zeros((LANES,), jnp.float32)
    a0, a1 = lax.fori_loop(0, n4, body4, (zero, zero))
    tail = n4 * UNROLL
    rvec = jnp.full((LANES,), r, dtype=jnp.int32)
    for j in range(UNROLL - 1):
        jv = jnp.full((LANES,), j, dtype=jnp.int32)
        mf = jnp.minimum(jnp.maximum(rvec - jv, 0), 1).astype(jnp.float32)
        a0 = a0 + rows[tail + j, pl.ds(0, LANES)] * mf
        a1 = a1 + rows[tail + j, pl.ds(LANES, LANES)] * mf
    nvec = jnp.full((LANES,), n, dtype=jnp.int32).astype(jnp.float32)
    out_v[b, pl.ds(0, LANES)] = a0 / nvec
    out_v[b, pl.ds(LANES, LANES)] = a1 / nvec


def _mean_embed_body(xs_hbm, len_hbm, w_hbm, out_hbm,
                     idx_all, rows_a, rows_b, len_v, out_v, sem_a, sem_b):
    c = lax.axis_index("c")
    s = lax.axis_index("s")
    wid = s * 2 + c
    base = wid * BPW

    pltpu.sync_copy(len_hbm.at[pl.ds(base, BPW)], len_v.at[pl.ds(0, BPW)])
    pltpu.sync_copy(xs_hbm.at[pl.ds(base, BPW), :], idx_all)

    _fire_batch(w_hbm, idx_all, rows_a, sem_a, len_v, 0)
    _fire_batch(w_hbm, idx_all, rows_b, sem_b, len_v, 1)

    def pair_body(k, carry):
        b0 = 2 * k
        _drain_batch(w_hbm, idx_all, rows_a, sem_a, len_v, b0)
        _accum_batch(rows_a, len_v, out_v, b0)
        _fire_batch(w_hbm, idx_all, rows_a, sem_a, len_v, b0 + 2)
        _drain_batch(w_hbm, idx_all, rows_b, sem_b, len_v, b0 + 1)
        _accum_batch(rows_b, len_v, out_v, b0 + 1)
        _fire_batch(w_hbm, idx_all, rows_b, sem_b, len_v, b0 + 3)
        return carry

    lax.fori_loop(0, BPW // 2 - 1, pair_body, 0)

    _drain_batch(w_hbm, idx_all, rows_a, sem_a, len_v, BPW - 2)
    _accum_batch(rows_a, len_v, out_v, BPW - 2)
    _drain_batch(w_hbm, idx_all, rows_b, sem_b, len_v, BPW - 1)
    _accum_batch(rows_b, len_v, out_v, BPW - 1)

    pltpu.sync_copy(out_v, out_hbm.at[pl.ds(base, BPW)])


@functools.partial(jax.jit, donate_argnums=())
def kernel(xs, xs_len, weight):
    mesh = plsc.VectorSubcoreMesh(core_axis_name="c", subcore_axis_name="s")
    k = functools.partial(
        pl.kernel,
        mesh=mesh,
        compiler_params=pltpu.CompilerParams(use_tc_tiling_on_sc=False),
        out_type=jax.ShapeDtypeStruct((B, D), jnp.float32),
        scratch_types=[
            pltpu.VMEM((BPW, L), jnp.int32),
            pltpu.VMEM((L, D), jnp.float32),
            pltpu.VMEM((L, D), jnp.float32),
            pltpu.VMEM((BPW + LANES,), jnp.int32),
            pltpu.VMEM((BPW, D), jnp.float32),
            pltpu.SemaphoreType.DMA,
            pltpu.SemaphoreType.DMA,
        ],
    )(_mean_embed_body)
    return k(xs.astype(jnp.int32), xs_len.astype(jnp.int32), weight)
